# Initial kernel scaffold; baseline (speedup 1.0000x reference)
import jax
import jax.numpy as jnp
from jax.experimental import pallas as pl


def kernel(pointcloud, params):
    raise NotImplementedError("write your pallas kernel here")


# speedup vs baseline: 4.1138x; 4.1138x over previous
import functools

import jax
import jax.numpy as jnp
from jax import lax
from jax.experimental import pallas as pl
from jax.experimental.pallas import tpu as pltpu

F32 = jnp.float32
HI = lax.Precision.HIGHEST
EPS = 1e-5


def _fiota(shape, dim):
    return lax.broadcasted_iota(jnp.int32, shape, dim).astype(F32)


def _bdot(a, b):
    return jnp.dot(
        a.astype(jnp.bfloat16), b.astype(jnp.bfloat16), preferred_element_type=F32
    )


def _fps_body(xs_ref, ys_ref, zs_ref, oxs_ref, oys_ref, ozs_ref, *, npoint, n):
    xs = xs_ref[...]
    ys = ys_ref[...]
    zs = zs_ref[...]
    b = xs.shape[0]
    lane = _fiota((b, n), 1)
    lane_np = _fiota((b, npoint), 1)

    def step(t, carry):
        dist, farf = carry
        oh = (lane == farf).astype(F32)
        cx = jnp.sum(oh * xs, axis=1, keepdims=True)
        cy = jnp.sum(oh * ys, axis=1, keepdims=True)
        cz = jnp.sum(oh * zs, axis=1, keepdims=True)
        sel = lane_np == t.astype(F32)
        oxs_ref[...] = jnp.where(sel, cx, oxs_ref[...])
        oys_ref[...] = jnp.where(sel, cy, oys_ref[...])
        ozs_ref[...] = jnp.where(sel, cz, ozs_ref[...])
        dx = xs - cx
        dy = ys - cy
        dz = zs - cz
        d = (dx * dx + dy * dy) + dz * dz
        dist = jnp.minimum(dist, d)
        m = jnp.max(dist, axis=1, keepdims=True)
        farf2 = jnp.min(jnp.where(dist == m, lane, float(n)), axis=1, keepdims=True)
        return dist, farf2

    dist0 = jnp.full((b, n), 1e10, F32)
    far0 = jnp.zeros((b, 1), F32)
    lax.fori_loop(0, npoint, step, (dist0, far0))


def _fps(xs, ys, zs, npoint):
    b, n = xs.shape
    out_shape = [jax.ShapeDtypeStruct((b, npoint), F32)] * 3
    return pl.pallas_call(
        functools.partial(_fps_body, npoint=npoint, n=n),
        out_shape=out_shape,
    )(xs, ys, zs)


def _bq_body(nx_ref, ny_ref, nz_ref, xs_ref, ys_ref, zs_ref, out_ref, *, n, r2, k):
    nx = nx_ref[0]
    ny = ny_ref[0]
    nz = nz_ref[0]
    xs = xs_ref[0]
    ys = ys_ref[0]
    zs = zs_ref[0]
    nsq = nx * nx + ny * ny + nz * nz
    psq = xs * xs + ys * ys + zs * zs
    bf = jnp.bfloat16
    nxb, nyb, nzb = (v.astype(bf).astype(F32) for v in (nx, ny, nz))
    xsb, ysb, zsb = (v.astype(bf).astype(F32) for v in (xs, ys, zs))
    dot = nxb * xsb + nyb * ysb + nzb * zsb
    sqr = (nsq + psq) - 2.0 * dot
    maskf = (sqr <= r2).astype(F32)

    lt = (
        lax.broadcasted_iota(jnp.int32, (128, 128), 0)
        <= lax.broadcasted_iota(jnp.int32, (128, 128), 1)
    ).astype(F32)
    cols = []
    run = jnp.zeros_like(nsq)
    for c in range(n // 128):
        seg = maskf[:, c * 128 : (c + 1) * 128]
        cs = jnp.dot(seg, lt, precision=HI)
        cols.append(cs + run)
        run = run + jnp.sum(seg, axis=1, keepdims=True)
    cnt = jnp.concatenate(cols, axis=1)

    outs = []
    for kk in range(k):
        outs.append(jnp.sum((cnt <= float(kk)).astype(F32), axis=1, keepdims=True))
    out = jnp.concatenate(outs, axis=1)
    first = out[:, :1]
    out_ref[0] = jnp.where(out >= float(n), first, out)


def _ball_query(cen, pts, r2, k, s_blk):
    b, s = cen[0].shape
    n = pts[0].shape[1]
    cen3 = [c.reshape(b, s, 1) for c in cen]
    pts3 = [p.reshape(b, 1, n) for p in pts]
    grid = (b, s // s_blk)
    cen_spec = pl.BlockSpec((1, s_blk, 1), lambda bi, si: (bi, si, 0))
    pts_spec = pl.BlockSpec((1, 1, n), lambda bi, si: (bi, 0, 0))
    return pl.pallas_call(
        functools.partial(_bq_body, n=n, r2=r2, k=k),
        grid=grid,
        in_specs=[cen_spec] * 3 + [pts_spec] * 3,
        out_specs=pl.BlockSpec((1, s_blk, k), lambda bi, si: (bi, si, 0)),
        out_shape=jax.ShapeDtypeStruct((b, s, k), F32),
    )(*cen3, *pts3)


def _norm_relu(y, ssum, ssq, m):
    mean = ssum * (1.0 / m)
    var = ssq * (1.0 / m) - mean * mean
    rs = lax.rsqrt(var + EPS)
    return jnp.maximum((y - mean) * rs, 0.0)


def _acc_stats(y, acc_ref, ssum_ref, ssq_ref, is_first, is_last):
    s = jnp.sum(y, axis=0, keepdims=True)
    sq = jnp.sum(y * y, axis=0, keepdims=True)

    @pl.when(is_first)
    def _():
        acc_ref[...] = jnp.zeros_like(acc_ref)

    acc_ref[0:1] += s
    acc_ref[1:2] += sq

    @pl.when(is_last)
    def _():
        ssum_ref[...] = acc_ref[0:1]
        ssq_ref[...] = acc_ref[1:2]


def _s1a_body(idx_ref, xyzp_ref, cenp_ref, w_ref, y_ref, ssum_ref, ssq_ref,
              acc_ref, *, n, cen_b, k):
    first = (pl.program_id(0) == 0) & (pl.program_id(1) == 0)
    last = (pl.program_id(0) == pl.num_programs(0) - 1) & (
        pl.program_id(1) == pl.num_programs(1) - 1
    )
    idx = idx_ref[0]
    iota2 = _fiota((cen_b * k, n), 1)
    oh = (iota2 == idx).astype(F32)
    g = jnp.dot(oh, xyzp_ref[0], precision=HI)
    cen = cenp_ref[0]
    gc = g - jnp.broadcast_to(cen[:, None, :], (cen_b, k, 8)).reshape(cen_b * k, 8)
    y = _bdot(gc, w_ref[...])
    y_ref[0] = y
    _acc_stats(y, acc_ref, ssum_ref, ssq_ref, first, last)


def _s1_layer_a(idx, xyzp, cenp, w1p, cen_b, k):
    b, sk, _ = idx.shape
    s = sk // k
    n = xyzp.shape[1]
    c1 = w1p.shape[1]
    grid = (b, s // cen_b)
    return pl.pallas_call(
        functools.partial(_s1a_body, n=n, cen_b=cen_b, k=k),
        grid=grid,
        in_specs=[
            pl.BlockSpec((1, cen_b * k, 1), lambda bi, si: (bi, si, 0)),
            pl.BlockSpec((1, n, 8), lambda bi, si: (bi, 0, 0)),
            pl.BlockSpec((1, cen_b, 8), lambda bi, si: (bi, si, 0)),
            pl.BlockSpec((8, c1), lambda bi, si: (0, 0)),
        ],
        out_specs=[
            pl.BlockSpec((1, cen_b * k, c1), lambda bi, si: (bi, si, 0)),
            pl.BlockSpec((1, c1), lambda bi, si: (0, 0)),
            pl.BlockSpec((1, c1), lambda bi, si: (0, 0)),
        ],
        out_shape=[
            jax.ShapeDtypeStruct((b, s * k, c1), F32),
            jax.ShapeDtypeStruct((1, c1), F32),
            jax.ShapeDtypeStruct((1, c1), F32),
        ],
        scratch_shapes=[pltpu.VMEM((2, c1), F32)],
    )(idx, xyzp, cenp, w1p)


def _s2a_body(idx_ref, f1_ref, xyzp_ref, cenp_ref, wa_ref, wb_ref, y_ref,
              ssum_ref, ssq_ref, acc_ref, *, n, cen_b, k):
    first = (pl.program_id(0) == 0) & (pl.program_id(1) == 0)
    last = (pl.program_id(0) == pl.num_programs(0) - 1) & (
        pl.program_id(1) == pl.num_programs(1) - 1
    )
    idx = idx_ref[0]
    iota2 = _fiota((cen_b * k, n), 1)
    oh = (iota2 == idx).astype(F32)
    gfeat = jnp.dot(oh, f1_ref[0], precision=HI)
    gxyz = jnp.dot(oh, xyzp_ref[0], precision=HI)
    cen = cenp_ref[0]
    gc = gxyz - jnp.broadcast_to(cen[:, None, :], (cen_b, k, 8)).reshape(cen_b * k, 8)
    y = _bdot(gc, wa_ref[...]) + _bdot(gfeat, wb_ref[...])
    y_ref[0] = y
    _acc_stats(y, acc_ref, ssum_ref, ssq_ref, first, last)


def _s2_layer_a(idx, f1, xyzp, cenp, wa, wb, cen_b, k):
    b, sk, _ = idx.shape
    s = sk // k
    n = f1.shape[1]
    c1 = f1.shape[2]
    c = wa.shape[1]
    grid = (b, s // cen_b)
    return pl.pallas_call(
        functools.partial(_s2a_body, n=n, cen_b=cen_b, k=k),
        grid=grid,
        in_specs=[
            pl.BlockSpec((1, cen_b * k, 1), lambda bi, si: (bi, si, 0)),
            pl.BlockSpec((1, n, c1), lambda bi, si: (bi, 0, 0)),
            pl.BlockSpec((1, n, 8), lambda bi, si: (bi, 0, 0)),
            pl.BlockSpec((1, cen_b, 8), lambda bi, si: (bi, si, 0)),
            pl.BlockSpec((8, c), lambda bi, si: (0, 0)),
            pl.BlockSpec((c1, c), lambda bi, si: (0, 0)),
        ],
        out_specs=[
            pl.BlockSpec((1, cen_b * k, c), lambda bi, si: (bi, si, 0)),
            pl.BlockSpec((1, c), lambda bi, si: (0, 0)),
            pl.BlockSpec((1, c), lambda bi, si: (0, 0)),
        ],
        out_shape=[
            jax.ShapeDtypeStruct((b, s * k, c), F32),
            jax.ShapeDtypeStruct((1, c), F32),
            jax.ShapeDtypeStruct((1, c), F32),
        ],
        scratch_shapes=[pltpu.VMEM((2, c), F32)],
    )(idx, f1, xyzp, cenp, wa, wb)


def _mid_body(y_ref, ssum_ref, ssq_ref, w_ref, o_ref, osum_ref, osq_ref,
              acc_ref, *, m):
    first = pl.program_id(0) == 0
    last = pl.program_id(0) == pl.num_programs(0) - 1
    x = _norm_relu(y_ref[...], ssum_ref[...], ssq_ref[...], m)
    y = _bdot(x, w_ref[...])
    o_ref[...] = y
    _acc_stats(y, acc_ref, osum_ref, osq_ref, first, last)


def _mid_layer(yprev, ssum, ssq, w, m_blk):
    m, c1 = yprev.shape
    c2 = w.shape[1]
    grid = (m // m_blk,)
    return pl.pallas_call(
        functools.partial(_mid_body, m=float(m)),
        grid=grid,
        in_specs=[
            pl.BlockSpec((m_blk, c1), lambda i: (i, 0)),
            pl.BlockSpec((1, c1), lambda i: (0, 0)),
            pl.BlockSpec((1, c1), lambda i: (0, 0)),
            pl.BlockSpec((c1, c2), lambda i: (0, 0)),
        ],
        out_specs=[
            pl.BlockSpec((m_blk, c2), lambda i: (i, 0)),
            pl.BlockSpec((1, c2), lambda i: (0, 0)),
            pl.BlockSpec((1, c2), lambda i: (0, 0)),
        ],
        out_shape=[
            jax.ShapeDtypeStruct((m, c2), F32),
            jax.ShapeDtypeStruct((1, c2), F32),
            jax.ShapeDtypeStruct((1, c2), F32),
        ],
        scratch_shapes=[pltpu.VMEM((2, c2), F32)],
    )(yprev, ssum, ssq, w)


def _last_body(y_ref, ssum_ref, ssq_ref, w_ref, o_ref, osum_ref, osq_ref,
               acc_ref, *, m, cen_b, k):
    first = pl.program_id(0) == 0
    last = pl.program_id(0) == pl.num_programs(0) - 1
    x = _norm_relu(y_ref[0], ssum_ref[...], ssq_ref[...], m)
    y = _bdot(x, w_ref[...])
    _acc_stats(y, acc_ref, osum_ref, osq_ref, first, last)
    c2 = y.shape[1]
    o_ref[0] = jnp.max(y.reshape(cen_b, k, c2), axis=1)


def _last_layer(yprev_g, ssum, ssq, w, cen_b, k):
    b, sk, c1 = yprev_g.shape
    s = sk // k
    c2 = w.shape[1]
    m = b * sk
    grid = (b * (s // cen_b),)
    nb = s // cen_b

    return pl.pallas_call(
        functools.partial(_last_body, m=float(m), cen_b=cen_b, k=k),
        grid=grid,
        in_specs=[
            pl.BlockSpec((1, cen_b * k, c1), lambda i: (i // nb, i % nb, 0)),
            pl.BlockSpec((1, c1), lambda i: (0, 0)),
            pl.BlockSpec((1, c1), lambda i: (0, 0)),
            pl.BlockSpec((c1, c2), lambda i: (0, 0)),
        ],
        out_specs=[
            pl.BlockSpec((1, cen_b, c2), lambda i: (i // nb, i % nb, 0)),
            pl.BlockSpec((1, c2), lambda i: (0, 0)),
            pl.BlockSpec((1, c2), lambda i: (0, 0)),
        ],
        out_shape=[
            jax.ShapeDtypeStruct((b, s, c2), F32),
            jax.ShapeDtypeStruct((1, c2), F32),
            jax.ShapeDtypeStruct((1, c2), F32),
        ],
        scratch_shapes=[pltpu.VMEM((2, c2), F32)],
    )(yprev_g, ssum, ssq, w)


def _fin_body(ym_ref, ssum_ref, ssq_ref, o_ref, *, m):
    o_ref[0] = _norm_relu(ym_ref[0], ssum_ref[...], ssq_ref[...], m)


def _finalize(ymax, ssum, ssq, m):
    b, s, c1 = ymax.shape
    return pl.pallas_call(
        functools.partial(_fin_body, m=float(m)),
        grid=(b,),
        in_specs=[
            pl.BlockSpec((1, s, c1), lambda i: (i, 0, 0)),
            pl.BlockSpec((1, c1), lambda i: (0, 0)),
            pl.BlockSpec((1, c1), lambda i: (0, 0)),
        ],
        out_specs=pl.BlockSpec((1, s, c1), lambda i: (i, 0, 0)),
        out_shape=jax.ShapeDtypeStruct((b, s, c1), F32),
    )(ymax, ssum, ssq)


def _s3_body(ym_ref, ssum_ref, ssq_ref, xyzp_ref, wa_ref, wb_ref, w2_ref,
             w3_ref, o_ref, *, m2, b, pts):
    feat = _norm_relu(ym_ref[...], ssum_ref[...], ssq_ref[...], m2)
    y1 = _bdot(xyzp_ref[...], wa_ref[...]) + _bdot(feat, wb_ref[...])

    def nr(y):
        mean = jnp.mean(y, axis=0, keepdims=True)
        var = jnp.mean((y - mean) ** 2, axis=0, keepdims=True)
        return jnp.maximum((y - mean) * lax.rsqrt(var + EPS), 0.0), mean, var

    x1, _, _ = nr(y1)
    y2 = _bdot(x1, w2_ref[...])
    x2, _, _ = nr(y2)
    y3 = _bdot(x2, w3_ref[...])
    mean3 = jnp.mean(y3, axis=0, keepdims=True)
    var3 = jnp.mean((y3 - mean3) ** 2, axis=0, keepdims=True)
    c3 = y3.shape[1]
    ymx = jnp.max(y3.reshape(b, pts, c3), axis=1)
    o_ref[...] = jnp.maximum((ymx - mean3) * lax.rsqrt(var3 + EPS), 0.0)


def _stage3(ymax2, ssum2, ssq2, xyzp2, wa, wb, w2, w3, m2, b, pts):
    c3 = w3.shape[1]
    mm = b * pts
    return pl.pallas_call(
        functools.partial(_s3_body, m2=float(m2), b=b, pts=pts),
        in_specs=[
            pl.BlockSpec(ymax2.shape, lambda: (0, 0)),
            pl.BlockSpec(ssum2.shape, lambda: (0, 0)),
            pl.BlockSpec(ssq2.shape, lambda: (0, 0)),
            pl.BlockSpec(xyzp2.shape, lambda: (0, 0)),
            pl.BlockSpec(wa.shape, lambda: (0, 0)),
            pl.BlockSpec(wb.shape, lambda: (0, 0)),
            pl.BlockSpec(w2.shape, lambda: (0, 0)),
            pl.BlockSpec(w3.shape, lambda: (0, 0)),
        ],
        out_specs=pl.BlockSpec((b, c3), lambda: (0, 0)),
        out_shape=jax.ShapeDtypeStruct((b, c3), F32),
    )(ymax2, ssum2, ssq2, xyzp2, wa, wb, w2, w3)


def kernel(pointcloud, params):
    b, n, _ = pointcloud.shape
    pc = pointcloud.astype(F32)
    xs, ys, zs = pc[:, :, 0], pc[:, :, 1], pc[:, :, 2]

    w1 = [params[0][i][0] for i in range(3)]
    w2 = [params[1][i][0] for i in range(3)]
    w3 = [params[2][i][0] for i in range(3)]

    s1, k1, r1 = 512, 64, 0.2
    nxs, nys, nzs = _fps(xs, ys, zs, s1)
    idx1 = _ball_query((nxs, nys, nzs), (xs, ys, zs), r1 * r1, k1, 128)

    zero_n = jnp.zeros((b, n, 1), F32)
    xyzp = jnp.concatenate(
        [xs[:, :, None], ys[:, :, None], zs[:, :, None]] + [zero_n] * 5, axis=2
    )
    zero_s = jnp.zeros((b, s1, 1), F32)
    cenp1 = jnp.concatenate(
        [nxs[:, :, None], nys[:, :, None], nzs[:, :, None]] + [zero_s] * 5, axis=2
    )
    w1ap = jnp.concatenate([w1[0], jnp.zeros((5, w1[0].shape[1]), F32)], axis=0)

    idx1f = idx1.reshape(b, s1 * k1, 1)
    y1, s1sum, s1sq = _s1_layer_a(idx1f, xyzp, cenp1, w1ap, cen_b=8, k=k1)
    m1 = b * s1 * k1
    y1f = y1.reshape(m1, -1)
    y2f, s2sum, s2sq = _mid_layer(y1f, s1sum, s1sq, w1[1], m_blk=4096)
    y2g = y2f.reshape(b, s1 * k1, -1)
    ymax1, s3sum, s3sq = _last_layer(y2g, s2sum, s2sq, w1[2], cen_b=32, k=k1)

    s2, k2, r2r = 128, 64, 0.4
    n2xs, n2ys, n2zs = _fps(nxs, nys, nzs, s2)
    idx2 = _ball_query((n2xs, n2ys, n2zs), (nxs, nys, nzs), r2r * r2r, k2, 128)

    w2a = jnp.concatenate([w2[0][:3], jnp.zeros((5, w2[0].shape[1]), F32)], axis=0)
    w2b = w2[0][3:]
    f1 = _finalize(ymax1, s3sum, s3sq, m1)

    zero_s2 = jnp.zeros((b, s2, 1), F32)
    cenp2 = jnp.concatenate(
        [n2xs[:, :, None], n2ys[:, :, None], n2zs[:, :, None]] + [zero_s2] * 5, axis=2
    )
    idx2f = idx2.reshape(b, s2 * k2, 1)
    y21, t1sum, t1sq = _s2_layer_a(idx2f, f1, cenp1, cenp2, w2a, w2b, cen_b=16, k=k2)
    m2 = b * s2 * k2
    y21f = y21.reshape(m2, -1)
    y22f, t2sum, t2sq = _mid_layer(y21f, t1sum, t1sq, w2[1], m_blk=4096)
    y22g = y22f.reshape(b, s2 * k2, -1)
    ymax2, t3sum, t3sq = _last_layer(y22g, t2sum, t2sq, w2[2], cen_b=32, k=k2)

    w3a = jnp.concatenate([w3[0][:3], jnp.zeros((5, w3[0].shape[1]), F32)], axis=0)
    w3b = w3[0][3:]
    xyzp2 = cenp2.reshape(b * s2, 8)
    ymax2f = ymax2.reshape(b * s2, -1)
    out = _stage3(ymax2f, t3sum, t3sq, xyzp2, w3a, w3b, w3[1], w3[2], m2, b, s2)
    return out
